# baseline (speedup 1.0000x reference)
import jax
import jax.numpy as jnp
from jax import lax
from jax.experimental import pallas as pl
from jax.experimental.pallas import tpu as pltpu

ALPHA = 3.0
K = 64
STRIP = 1024
N_ITERS = 8


def _dense(e, w, b):
    h = lax.dot_general(e, w, (((1,), (1,)), ((), ())),
                        preferred_element_type=jnp.float32) + b
    return jnp.tanh(ALPHA * h)


def _t_block(v1s, v2full):
    t1 = lax.dot_general(v1s, v2full, (((1,), (1,)), ((), ())),
                         preferred_element_type=jnp.float32)
    t2 = lax.dot_general(v2full, v1s, (((1,), (1,)), ((), ())),
                         preferred_element_type=jnp.float32)
    t1_16 = t1.astype(jnp.bfloat16)
    t2_16 = t2.astype(jnp.bfloat16)
    return jnp.tanh(t1_16 - t2_16.T)


def _count_gt(c16, mid16):
    c = jnp.where(c16 > mid16, jnp.bfloat16(1.0), jnp.bfloat16(0.0))
    w = c.shape[1]
    while w > 128:
        w //= 2
        c = c[:, :w] + c[:, w:2 * w]
    return jnp.sum(c.astype(jnp.float32), axis=1, keepdims=True)


def _fused_kernel(e1_ref, w1_ref, b1_ref, e2_ref, w2_ref, b2_ref,
                  o_ref, v1_s, v2_s):
    strips_per_mod = e1_ref.shape[1] // STRIP
    s = pl.program_id(0)

    @pl.when(s == 0)
    def _():
        for b in range(e1_ref.shape[0]):
            v1_s[b] = (ALPHA * _dense(e1_ref[b], w1_ref[b], b1_ref[b, 0])
                       ).astype(jnp.bfloat16)
            v2_s[b] = _dense(e2_ref[b], w2_ref[b], b2_ref[b, 0]
                             ).astype(jnp.bfloat16)

    i = s // strips_per_mod
    r = s % strips_per_mod
    rows = pl.ds(r * STRIP, STRIP)
    t16 = jnp.concatenate(
        [_t_block(v1_s[2 * i, rows, :], v2_s[2 * i]),
         _t_block(v1_s[2 * i + 1, rows, :], v2_s[2 * i + 1])],
        axis=1)

    lo = jnp.full((STRIP, 1), -(2.0 ** -9), jnp.float32)
    hi = jnp.full((STRIP, 1), 1.0, jnp.float32)
    for _ in range(N_ITERS):
        mid = 0.5 * (lo + hi)
        cnt = _count_gt(t16, mid.astype(jnp.bfloat16))
        ge_k = cnt >= K
        lo = jnp.where(ge_k, mid, lo)
        hi = jnp.where(ge_k, hi, mid)
    thr16 = jnp.maximum(lo, 0.0).astype(jnp.bfloat16)
    o_ref[...] = jnp.where(t16 > thr16, t16,
                           jnp.bfloat16(0.0)).astype(jnp.float32)


def kernel(idx, emb1, emb2, W1, b1, W2, b2):
    nm, n_sub, dim = emb1.shape
    n_mod = 2
    N = n_mod * n_sub
    whole = lambda s: (0, 0, 0)

    out = pl.pallas_call(
        _fused_kernel,
        out_shape=jax.ShapeDtypeStruct((N, N), jnp.float32),
        grid=(n_mod * (n_sub // STRIP),),
        in_specs=[
            pl.BlockSpec((nm, n_sub, dim), whole),
            pl.BlockSpec((nm, dim, dim), whole),
            pl.BlockSpec((nm, 1, dim), whole),
            pl.BlockSpec((nm, n_sub, dim), whole),
            pl.BlockSpec((nm, dim, dim), whole),
            pl.BlockSpec((nm, 1, dim), whole),
        ],
        out_specs=pl.BlockSpec((STRIP, N), lambda s: (s, 0)),
        scratch_shapes=[
            pltpu.VMEM((nm, n_sub, dim), jnp.bfloat16),
            pltpu.VMEM((nm, n_sub, dim), jnp.bfloat16),
        ],
    )(emb1, W1, b1.reshape(nm, 1, dim), emb2, W2, b2.reshape(nm, 1, dim))
    return out


# speedup vs baseline: 1.0554x; 1.0554x over previous
import jax
import jax.numpy as jnp
from jax import lax
from jax.experimental import pallas as pl
from jax.experimental.pallas import tpu as pltpu

ALPHA = 3.0
K = 64
STRIP = 512
N_ITERS = 8


def _dense(e, w, b):
    h = lax.dot_general(e, w, (((1,), (1,)), ((), ())),
                        preferred_element_type=jnp.float32) + b
    return jnp.tanh(ALPHA * h)


def _t_block(v1s, v2full):
    t1 = lax.dot_general(v1s, v2full, (((1,), (1,)), ((), ())),
                         preferred_element_type=jnp.float32)
    t2 = lax.dot_general(v2full, v1s, (((1,), (1,)), ((), ())),
                         preferred_element_type=jnp.float32)
    t1_16 = t1.astype(jnp.bfloat16)
    t2_16 = t2.astype(jnp.bfloat16)
    return jnp.tanh(t1_16 - t2_16.T)


def _count_gt(c16, mid16):
    c = jnp.where(c16 > mid16, jnp.bfloat16(1.0), jnp.bfloat16(0.0))
    w = c.shape[1]
    while w > 128:
        w //= 2
        c = c[:, :w] + c[:, w:2 * w]
    return jnp.sum(c.astype(jnp.float32), axis=1, keepdims=True)


def _fused_kernel(e1_ref, w1_ref, b1_ref, e2_ref, w2_ref, b2_ref,
                  o_ref, v1_s, v2_s):
    strips_per_mod = e1_ref.shape[1] // STRIP
    s = pl.program_id(0)

    @pl.when(s == 0)
    def _():
        for b in range(e1_ref.shape[0]):
            v1_s[b] = (ALPHA * _dense(e1_ref[b], w1_ref[b], b1_ref[b, 0])
                       ).astype(jnp.bfloat16)
            v2_s[b] = _dense(e2_ref[b], w2_ref[b], b2_ref[b, 0]
                             ).astype(jnp.bfloat16)

    i = s // strips_per_mod
    r = s % strips_per_mod
    rows = pl.ds(r * STRIP, STRIP)
    t16 = jnp.concatenate(
        [_t_block(v1_s[2 * i, rows, :], v2_s[2 * i]),
         _t_block(v1_s[2 * i + 1, rows, :], v2_s[2 * i + 1])],
        axis=1)

    lo = jnp.full((STRIP, 1), -(2.0 ** -9), jnp.float32)
    hi = jnp.full((STRIP, 1), 1.0, jnp.float32)
    for _ in range(N_ITERS):
        mid = 0.5 * (lo + hi)
        cnt = _count_gt(t16, mid.astype(jnp.bfloat16))
        ge_k = cnt >= K
        lo = jnp.where(ge_k, mid, lo)
        hi = jnp.where(ge_k, hi, mid)
    thr16 = jnp.maximum(lo, 0.0).astype(jnp.bfloat16)
    o_ref[...] = jnp.where(t16 > thr16, t16,
                           jnp.bfloat16(0.0)).astype(jnp.float32)


def kernel(idx, emb1, emb2, W1, b1, W2, b2):
    nm, n_sub, dim = emb1.shape
    n_mod = 2
    N = n_mod * n_sub
    whole = lambda s: (0, 0, 0)

    out = pl.pallas_call(
        _fused_kernel,
        out_shape=jax.ShapeDtypeStruct((N, N), jnp.float32),
        grid=(n_mod * (n_sub // STRIP),),
        in_specs=[
            pl.BlockSpec((nm, n_sub, dim), whole),
            pl.BlockSpec((nm, dim, dim), whole),
            pl.BlockSpec((nm, 1, dim), whole),
            pl.BlockSpec((nm, n_sub, dim), whole),
            pl.BlockSpec((nm, dim, dim), whole),
            pl.BlockSpec((nm, 1, dim), whole),
        ],
        out_specs=pl.BlockSpec((STRIP, N), lambda s: (s, 0)),
        scratch_shapes=[
            pltpu.VMEM((nm, n_sub, dim), jnp.bfloat16),
            pltpu.VMEM((nm, n_sub, dim), jnp.bfloat16),
        ],
    )(emb1, W1, b1.reshape(nm, 1, dim), emb2, W2, b2.reshape(nm, 1, dim))
    return out
